# baseline (speedup 1.0000x reference)
import functools

import jax
import jax.numpy as jnp
from jax import lax
from jax.experimental import pallas as pl
from jax.experimental.pallas import tpu as pltpu
from jax.experimental.pallas import tpu_sc as plsc

B = 16384
F = 26
V = 100000
D = 16

_INFO = plsc.get_sparse_core_info()
NC = _INFO.num_cores
NS = _INFO.num_subcores
L = _INFO.num_lanes
NW = NC * NS

VB = 1408
VMAIN = V - (V % 128)
NSLAB = F * (VMAIN // VB)
WPF = 12504
VPAD = WPF * 128 // D
TAILW = F * (V % 128) * D // 128

RW = B // NW
RC = 128
NCH = RW // RC
CN = RC * F

_mesh = plsc.VectorSubcoreMesh(core_axis_name="c", subcore_axis_name="s")


@functools.partial(
    pl.kernel,
    mesh=_mesh,
    out_type=jax.ShapeDtypeStruct((F, WPF, 128), jnp.float32),
    scratch_types=[
        pltpu.VMEM((D, VB), jnp.float32),
        pltpu.VMEM((VB * D // 128, 128), jnp.float32),
        pltpu.VMEM((8, 128), jnp.float32),
    ],
    compiler_params=pltpu.CompilerParams(
        use_tc_tiling_on_sc=True, needs_layout_passes=False
    ),
)
def _transpose_table(tdv_hbm, tail_hbm, w_hbm, slab_v, tslab_v, tail_v):
    wid = lax.axis_index("s") * NC + lax.axis_index("c")
    lane = lax.iota(jnp.int32, L)

    def slab_body(k, _):
        sid = wid + NW * k

        @pl.when(sid < NSLAB)
        def _():
            fi = sid // (VMAIN // VB)
            v0 = pl.multiple_of((sid % (VMAIN // VB)) * VB, 128)
            pltpu.sync_copy(tdv_hbm.at[fi, :, pl.ds(v0, VB)], slab_v)

            def col_body(v, _):
                x = plsc.load_gather(slab_v, [lane, jnp.full((L,), v, jnp.int32)])
                tslab_v[v // 8, pl.ds((v % 8) * D, D)] = x
                return ()

            lax.fori_loop(0, VB, col_body, ())

            wr0 = pl.multiple_of(v0 * D // 128, 8)
            pltpu.sync_copy(tslab_v, w_hbm.at[fi, pl.ds(wr0, VB * D // 128)])
        return ()

    lax.fori_loop(0, (NSLAB + NW - 1) // NW, slab_body, ())

    @pl.when(wid == 0)
    def _():
        def tail_body(fi, _):
            pltpu.sync_copy(tail_hbm.at[fi], tail_v)
            pltpu.sync_copy(tail_v, w_hbm.at[fi, pl.ds(VMAIN * D // 128, 8)])
            return ()

        lax.fori_loop(0, F, tail_body, ())


@functools.partial(
    pl.kernel,
    mesh=_mesh,
    out_type=jax.ShapeDtypeStruct((B * F, D), jnp.float32),
    scratch_types=[
        pltpu.VMEM((RC, F), jnp.int32),
        pltpu.VMEM((CN,), jnp.int32),
        pltpu.VMEM((CN, D), jnp.float32),
        pltpu.SemaphoreType.DMA,
    ],
    compiler_params=pltpu.CompilerParams(
        use_tc_tiling_on_sc=False, needs_layout_passes=False
    ),
)
def _gather_rows(table_hbm, idx_hbm, out_hbm, idx_v, ids_v, rows_v, sem):
    wid = lax.axis_index("s") * NC + lax.axis_index("c")
    row0 = wid * RW

    off_lo = lax.iota(jnp.int32, L) * VPAD
    off_hi = (lax.iota(jnp.int32, L) + (F - L)) * VPAD

    def chunk_body(i, _):
        b0 = row0 + i * RC
        pltpu.sync_copy(idx_hbm.at[pl.ds(b0, RC), :], idx_v)

        def row_body(r, _):
            ids_v[pl.ds(r * F, L)] = idx_v[r, pl.ds(0, L)] + off_lo
            ids_v[pl.ds(r * F + (F - L), L)] = idx_v[r, pl.ds(F - L, L)] + off_hi
            return ()

        lax.fori_loop(0, RC, row_body, ())

        pltpu.async_copy(table_hbm.at[ids_v], rows_v, sem).wait()
        pltpu.sync_copy(rows_v, out_hbm.at[pl.ds(b0 * F, CN)])
        return ()

    lax.fori_loop(0, NCH, chunk_body, ())


def kernel(indices, tables):
    tdv = tables.transpose(0, 2, 1)
    tail = jnp.concatenate(
        [
            tables[:, VMAIN:, :].reshape(F, 4, 128),
            jnp.zeros((F, 4, 128), jnp.float32),
        ],
        axis=1,
    )
    w = _transpose_table(tdv, tail)
    out = _gather_rows(w.reshape(F * VPAD, D), indices)
    return out.reshape(B, F * D)


# speedup vs baseline: 2.1957x; 2.1957x over previous
import functools

import jax
import jax.numpy as jnp
from jax import lax
from jax.experimental import pallas as pl
from jax.experimental.pallas import tpu as pltpu
from jax.experimental.pallas import tpu_sc as plsc

B = 16384
F = 26
V = 100000
D = 16

_INFO = plsc.get_sparse_core_info()
NC = _INFO.num_cores
NS = _INFO.num_subcores
L = _INFO.num_lanes
NW = NC * NS

VB = 1408
VMAIN = V - (V % 128)
NSLAB = F * (VMAIN // VB)
WPF = 12504
VPAD = WPF * 128 // D
TAILW = F * (V % 128) * D // 128

RW = B // NW
RC = 128
NCH = RW // RC
CN = RC * F

_mesh = plsc.VectorSubcoreMesh(core_axis_name="c", subcore_axis_name="s")


@functools.partial(
    pl.kernel,
    mesh=_mesh,
    out_type=jax.ShapeDtypeStruct((F, WPF, 128), jnp.float32),
    scratch_types=[
        pltpu.VMEM((D, VB), jnp.float32),
        pltpu.VMEM((VB * D // 128, 128), jnp.float32),
        pltpu.VMEM((8, 128), jnp.float32),
    ],
    compiler_params=pltpu.CompilerParams(
        use_tc_tiling_on_sc=True, needs_layout_passes=False
    ),
)
def _transpose_table(tdv_hbm, tail_hbm, w_hbm, slab_v, tslab_v, tail_v):
    wid = lax.axis_index("s") * NC + lax.axis_index("c")
    lane = lax.iota(jnp.int32, L)
    row_base = lane // 8
    col_bases = [(lane % 8) * D + d for d in range(D)]

    def slab_body(k, _):
        sid = wid + NW * k

        @pl.when(sid < NSLAB)
        def _():
            fi = sid // (VMAIN // VB)
            v0 = pl.multiple_of((sid % (VMAIN // VB)) * VB, 128)
            pltpu.sync_copy(tdv_hbm.at[fi, :, pl.ds(v0, VB)], slab_v)

            def chunk_body(c, _):
                rows = row_base + c * 2
                c16 = c * L
                for d in range(D):
                    x = slab_v[d, pl.ds(c16, L)]
                    plsc.store_scatter(tslab_v, [rows, col_bases[d]], x)
                return ()

            lax.fori_loop(0, VB // L, chunk_body, ())

            wr0 = pl.multiple_of(v0 * D // 128, 8)
            pltpu.sync_copy(tslab_v, w_hbm.at[fi, pl.ds(wr0, VB * D // 128)])
        return ()

    lax.fori_loop(0, (NSLAB + NW - 1) // NW, slab_body, ())

    @pl.when(wid == 0)
    def _():
        def tail_body(fi, _):
            pltpu.sync_copy(tail_hbm.at[fi], tail_v)
            pltpu.sync_copy(tail_v, w_hbm.at[fi, pl.ds(VMAIN * D // 128, 8)])
            return ()

        lax.fori_loop(0, F, tail_body, ())


@functools.partial(
    pl.kernel,
    mesh=_mesh,
    out_type=jax.ShapeDtypeStruct((B * F, D), jnp.float32),
    scratch_types=[
        pltpu.VMEM((RC, F), jnp.int32),
        pltpu.VMEM((CN,), jnp.int32),
        pltpu.VMEM((CN, D), jnp.float32),
        pltpu.SemaphoreType.DMA,
    ],
    compiler_params=pltpu.CompilerParams(
        use_tc_tiling_on_sc=False, needs_layout_passes=False
    ),
)
def _gather_rows(table_hbm, idx_hbm, out_hbm, idx_v, ids_v, rows_v, sem):
    wid = lax.axis_index("s") * NC + lax.axis_index("c")
    row0 = wid * RW

    off_lo = lax.iota(jnp.int32, L) * VPAD
    off_hi = (lax.iota(jnp.int32, L) + (F - L)) * VPAD

    def chunk_body(i, _):
        b0 = row0 + i * RC
        pltpu.sync_copy(idx_hbm.at[pl.ds(b0, RC), :], idx_v)

        def row_body(r, _):
            ids_v[pl.ds(r * F, L)] = idx_v[r, pl.ds(0, L)] + off_lo
            ids_v[pl.ds(r * F + (F - L), L)] = idx_v[r, pl.ds(F - L, L)] + off_hi
            return ()

        lax.fori_loop(0, RC, row_body, ())

        pltpu.async_copy(table_hbm.at[ids_v], rows_v, sem).wait()
        pltpu.sync_copy(rows_v, out_hbm.at[pl.ds(b0 * F, CN)])
        return ()

    lax.fori_loop(0, NCH, chunk_body, ())


def kernel(indices, tables):
    tdv = tables.transpose(0, 2, 1)
    tail = jnp.concatenate(
        [
            tables[:, VMAIN:, :].reshape(F, 4, 128),
            jnp.zeros((F, 4, 128), jnp.float32),
        ],
        axis=1,
    )
    w = _transpose_table(tdv, tail)
    out = _gather_rows(w.reshape(F * VPAD, D), indices)
    return out.reshape(B, F * D)
